# baseline (speedup 1.0000x reference)
import functools

import jax
import jax.numpy as jnp
from jax import lax
from jax.experimental import pallas as pl
from jax.experimental.pallas import tpu as pltpu
from jax.experimental.pallas import tpu_sc as plsc

N, E, D = 10000, 320000, 128
NC, NS = 2, 16
NW = NC * NS
L = 16
NSL = 640
XC = 80
NCH = NSL // XC
EC = E // NW
ECA = E // NS
_MESH = plsc.VectorSubcoreMesh(core_axis_name="c", subcore_axis_name="s")
_PARAMS = pltpu.CompilerParams(needs_layout_passes=False)


@functools.partial(
    pl.kernel,
    out_type=(
        jax.ShapeDtypeStruct((E,), jnp.float32),
        jax.ShapeDtypeStruct((NC * NS * N,), jnp.float32),
    ),
    mesh=_MESH,
    compiler_params=_PARAMS,
    scratch_types=(
        pltpu.VMEM((2 * XC * D,), jnp.float32),
        pltpu.VMEM((D,), jnp.float32),
        pltpu.VMEM((L,), jnp.float32),
        pltpu.VMEM((NSL,), jnp.float32),
        pltpu.VMEM((ECA,), jnp.int32),
        pltpu.VMEM((N,), jnp.float32),
        pltpu.VMEM((NS * NSL,), jnp.float32),
        pltpu.VMEM((NSL,), jnp.float32),
        pltpu.VMEM((N,), jnp.float32),
        pltpu.VMEM((N,), jnp.float32),
        pltpu.VMEM((EC,), jnp.int32),
        pltpu.VMEM((EC,), jnp.int32),
        pltpu.VMEM((EC,), jnp.float32),
        pltpu.VMEM((EC,), jnp.float32),
        pltpu.VMEM_SHARED((N,), jnp.float32),
        pltpu.VMEM_SHARED((N,), jnp.float32),
        pltpu.SemaphoreType.DMA,
        pltpu.SemaphoreType.DMA,
        pltpu.SemaphoreType.DMA,
        pltpu.SemaphoreType.DMA,
    ),
)
def _node_attention_kernel(x_hbm, row_hbm, col_hbm, ea_hbm, w_hbm, b_hbm,
                           out_hbm, hist_hbm,
                           x_v, w_v, b_v, z_v, colh_v, hist_v, hsl_v, dsl_v,
                           diag_t, dinv_t, row_v, cole_v, ea_v, out_v,
                           diag_sh, dinv_sh,
                           sem_x, sem_e, sem_h, sem_t):
    cid = lax.axis_index("c")
    sid = lax.axis_index("s")
    wid = cid * NS + sid
    nbase = pl.multiple_of(jnp.minimum(sid * NSL, N - NSL), 8)
    eoff = wid * EC

    edge_copies = (
        pltpu.make_async_copy(row_hbm.at[pl.ds(eoff, EC)], row_v, sem_e),
        pltpu.make_async_copy(col_hbm.at[pl.ds(eoff, EC)], cole_v, sem_e),
        pltpu.make_async_copy(ea_hbm.at[pl.ds(eoff, EC)], ea_v, sem_e),
    )
    for c in edge_copies:
        c.start()

    x_chunk_copies = tuple(
        pltpu.make_async_copy(
            x_hbm.at[pl.ds((nbase + c * XC) * D, XC * D)],
            x_v.at[pl.ds((c % 2) * XC * D, XC * D)],
            sem_x,
        )
        for c in range(NCH)
    )
    x_chunk_copies[0].start()

    pltpu.sync_copy(col_hbm.at[pl.ds(sid * ECA, ECA)], colh_v)
    pltpu.sync_copy(w_hbm, w_v)
    pltpu.sync_copy(b_hbm, b_v)

    def fill_zero(k, _):
        hist_v[pl.ds(k * L, L)] = jnp.zeros((L,), jnp.float32)
        return 0

    lax.fori_loop(0, N // L, fill_zero, 0, unroll=8)

    one16 = jnp.full((L,), 1.0, jnp.float32)

    def hist_body(k, _):
        plsc.addupdate_scatter(hist_v, [colh_v[pl.ds(k * L, L)]], one16)
        return 0

    with jax.named_scope("ph_hist"):
        lax.fori_loop(0, ECA // L, hist_body, 0, unroll=16)

    pltpu.sync_copy(hist_v, hist_hbm.at[pl.ds(wid * N, N)])

    wregs = [w_v[pl.ds(d8 * L, L)] for d8 in range(D // L)]
    lane = jnp.arange(L, dtype=jnp.int32)

    with jax.named_scope("ph_dot"):
        for c in range(NCH):
            x_chunk_copies[c].wait()
            if c + 1 < NCH:
                x_chunk_copies[c + 1].start()
            cbase = (c % 2) * XC * D

            def group_body(g, _, _cbase=cbase, _c=c):
                zvec = jnp.zeros((L,), jnp.float32)
                for j in range(L):
                    off = _cbase + (g * L + j) * D
                    acc0 = x_v[pl.ds(off, L)] * wregs[0]
                    acc1 = x_v[pl.ds(off + L, L)] * wregs[1]
                    for d8 in range(2, D // L, 2):
                        acc0 = acc0 + x_v[pl.ds(off + d8 * L, L)] * wregs[d8]
                        acc1 = (acc1
                                + x_v[pl.ds(off + (d8 + 1) * L, L)]
                                * wregs[d8 + 1])
                    zvec = jnp.where(lane == j, jnp.sum(acc0 + acc1), zvec)
                z_v[pl.ds(_c * XC + g * L, L)] = zvec
                return 0

            lax.fori_loop(0, XC // L, group_body, 0)

    def sig_body(j, _):
        zv = z_v[pl.ds(j * L, L)] + b_v[...]
        z_v[pl.ds(j * L, L)] = 1.0 / (1.0 + jnp.exp(-zv))
        return 0

    lax.fori_loop(0, NSL // L, sig_body, 0, unroll=4)
    pltpu.sync_copy(z_v, diag_sh.at[pl.ds(nbase, NSL)])

    plsc.subcore_barrier()

    slice_copies = tuple(
        pltpu.make_async_copy(
            hist_hbm.at[pl.ds((cid * NS + t) * N + nbase, NSL)],
            hsl_v.at[pl.ds(t * NSL, NSL)],
            sem_h,
        )
        for t in range(NS)
    )
    for c in slice_copies:
        c.start()
    for c in slice_copies:
        c.wait()

    def red_body(k, _):
        acc = [
            hsl_v[pl.ds(t * NSL + k * L, L)]
            + hsl_v[pl.ds((t + 1) * NSL + k * L, L)]
            for t in range(0, NS, 2)
        ]
        acc = [acc[t] + acc[t + 1] for t in range(0, 8, 2)]
        acc = [acc[t] + acc[t + 1] for t in range(0, 4, 2)]
        dsl_v[pl.ds(k * L, L)] = 1.0 / (acc[0] + acc[1])
        return 0

    lax.fori_loop(0, NSL // L, red_body, 0, unroll=4)
    pltpu.sync_copy(dsl_v, dinv_sh.at[pl.ds(nbase, NSL)])

    plsc.subcore_barrier()

    table_copies = (
        pltpu.make_async_copy(diag_sh, diag_t, sem_t),
        pltpu.make_async_copy(dinv_sh, dinv_t, sem_t),
    )
    for c in table_copies:
        c.start()
    for c in table_copies:
        c.wait()
    for c in edge_copies:
        c.wait()

    def edge_body(i, _):
        s = pl.ds(i * L, L)
        dv = plsc.load_gather(dinv_t, [row_v[s]])
        gv = plsc.load_gather(diag_t, [cole_v[s]])
        out_v[s] = ea_v[s] * dv * gv
        return 0

    with jax.named_scope("ph_edge"):
        lax.fori_loop(0, EC // L, edge_body, 0, unroll=8)
    pltpu.sync_copy(out_v, out_hbm.at[pl.ds(eoff, EC)])


def kernel(x, edge_index, edge_attr, W, b):
    row = edge_index[0]
    col = edge_index[1]
    w_flat = W.reshape(D)
    x_flat = x.reshape(N * D)
    b_vec = jnp.broadcast_to(b.reshape(1), (L,)).astype(jnp.float32)
    adj_val, _ = _node_attention_kernel(x_flat, row, col, edge_attr,
                                        w_flat, b_vec)
    return (edge_index, adj_val)


# speedup vs baseline: 1.1939x; 1.1939x over previous
import functools

import jax
import jax.numpy as jnp
from jax import lax
from jax.experimental import pallas as pl
from jax.experimental.pallas import tpu as pltpu
from jax.experimental.pallas import tpu_sc as plsc

N, E, D = 10000, 320000, 128
NC, NS = 2, 16
NW = NC * NS
L = 16
NSL = 640
XC = 80
NCH = NSL // XC
EC = E // NW
ECA = E // NS
_MESH = plsc.VectorSubcoreMesh(core_axis_name="c", subcore_axis_name="s")
_PARAMS = pltpu.CompilerParams(needs_layout_passes=False)


@functools.partial(
    pl.kernel,
    out_type=(
        jax.ShapeDtypeStruct((E,), jnp.float32),
        jax.ShapeDtypeStruct((NC * NS * N,), jnp.float32),
    ),
    mesh=_MESH,
    compiler_params=_PARAMS,
    scratch_types=(
        pltpu.VMEM((2 * XC * D,), jnp.float32),
        pltpu.VMEM((D,), jnp.float32),
        pltpu.VMEM((L,), jnp.float32),
        pltpu.VMEM((NSL,), jnp.float32),
        pltpu.VMEM((ECA,), jnp.int32),
        pltpu.VMEM((N,), jnp.float32),
        pltpu.VMEM((NS * NSL,), jnp.float32),
        pltpu.VMEM((NSL,), jnp.float32),
        pltpu.VMEM((N,), jnp.float32),
        pltpu.VMEM((N,), jnp.float32),
        pltpu.VMEM((EC,), jnp.int32),
        pltpu.VMEM((EC,), jnp.int32),
        pltpu.VMEM((EC,), jnp.float32),
        pltpu.VMEM((EC,), jnp.float32),
        pltpu.VMEM_SHARED((N,), jnp.float32),
        pltpu.VMEM_SHARED((N,), jnp.float32),
        pltpu.SemaphoreType.DMA,
        pltpu.SemaphoreType.DMA,
        pltpu.SemaphoreType.DMA,
        pltpu.SemaphoreType.DMA,
        pltpu.SemaphoreType.DMA,
    ),
)
def _node_attention_kernel(x_hbm, ei_hbm, ea_hbm, w_hbm, b_hbm,
                           out_hbm, hist_hbm,
                           x_v, w_v, b_v, z_v, colh_v, hist_v, hsl_v, dsl_v,
                           diag_t, dinv_t, row_v, cole_v, ea_v, out_v,
                           diag_sh, dinv_sh,
                           sem_x, sem_e, sem_h, sem_t, sem_c):
    cid = lax.axis_index("c")
    sid = lax.axis_index("s")
    wid = cid * NS + sid
    nbase = pl.multiple_of(jnp.minimum(sid * NSL, N - NSL), 8)
    eoff = wid * EC

    edge_copies = (
        pltpu.make_async_copy(ei_hbm.at[pl.ds(eoff, EC)], row_v, sem_e),
        pltpu.make_async_copy(ei_hbm.at[pl.ds(E + eoff, EC)], cole_v, sem_e),
        pltpu.make_async_copy(ea_hbm.at[pl.ds(eoff, EC)], ea_v, sem_e),
    )
    for c in edge_copies:
        c.start()

    x_chunk_copies = tuple(
        pltpu.make_async_copy(
            x_hbm.at[pl.ds((nbase + c * XC) * D, XC * D)],
            x_v.at[pl.ds((c % 2) * XC * D, XC * D)],
            sem_x,
        )
        for c in range(NCH)
    )
    x_chunk_copies[0].start()

    hc = pltpu.async_copy(ei_hbm.at[pl.ds(E + sid * ECA, ECA)], colh_v,
                          sem_c)
    pltpu.sync_copy(w_hbm, w_v)
    pltpu.sync_copy(b_hbm, b_v)

    def fill_zero(k, _):
        hist_v[pl.ds(k * L, L)] = jnp.zeros((L,), jnp.float32)
        return 0

    lax.fori_loop(0, N // L, fill_zero, 0, unroll=8)

    one16 = jnp.full((L,), 1.0, jnp.float32)

    def hist_body(k, _):
        plsc.addupdate_scatter(hist_v, [colh_v[pl.ds(k * L, L)]], one16)
        return 0

    hc.wait()
    with jax.named_scope("ph_hist"):
        lax.fori_loop(0, ECA // L, hist_body, 0, unroll=16)

    pltpu.sync_copy(hist_v, hist_hbm.at[pl.ds(wid * N, N)])
    plsc.subcore_barrier()

    slice_copies = tuple(
        pltpu.make_async_copy(
            hist_hbm.at[pl.ds((cid * NS + t) * N + nbase, NSL)],
            hsl_v.at[pl.ds(t * NSL, NSL)],
            sem_h,
        )
        for t in range(NS)
    )
    for c in slice_copies:
        c.start()

    wregs = [w_v[pl.ds(d8 * L, L)] for d8 in range(D // L)]
    lane = jnp.arange(L, dtype=jnp.int32)

    with jax.named_scope("ph_dot"):
        for c in range(NCH):
            x_chunk_copies[c].wait()
            if c + 1 < NCH:
                x_chunk_copies[c + 1].start()
            cbase = (c % 2) * XC * D

            def group_body(g, _, _cbase=cbase, _c=c):
                zvec = jnp.zeros((L,), jnp.float32)
                for j in range(L):
                    off = _cbase + (g * L + j) * D
                    acc0 = x_v[pl.ds(off, L)] * wregs[0]
                    acc1 = x_v[pl.ds(off + L, L)] * wregs[1]
                    for d8 in range(2, D // L, 2):
                        acc0 = acc0 + x_v[pl.ds(off + d8 * L, L)] * wregs[d8]
                        acc1 = (acc1
                                + x_v[pl.ds(off + (d8 + 1) * L, L)]
                                * wregs[d8 + 1])
                    zvec = jnp.where(lane == j, jnp.sum(acc0 + acc1), zvec)
                z_v[pl.ds(_c * XC + g * L, L)] = zvec
                return 0

            lax.fori_loop(0, XC // L, group_body, 0)

    def sig_body(j, _):
        zv = z_v[pl.ds(j * L, L)] + b_v[...]
        z_v[pl.ds(j * L, L)] = 1.0 / (1.0 + jnp.exp(-zv))
        return 0

    lax.fori_loop(0, NSL // L, sig_body, 0, unroll=4)
    pltpu.sync_copy(z_v, diag_sh.at[pl.ds(nbase, NSL)])

    for c in slice_copies:
        c.wait()

    def red_body(k, _):
        acc = [
            hsl_v[pl.ds(t * NSL + k * L, L)]
            + hsl_v[pl.ds((t + 1) * NSL + k * L, L)]
            for t in range(0, NS, 2)
        ]
        acc = [acc[t] + acc[t + 1] for t in range(0, 8, 2)]
        acc = [acc[t] + acc[t + 1] for t in range(0, 4, 2)]
        dsl_v[pl.ds(k * L, L)] = 1.0 / (acc[0] + acc[1])
        return 0

    lax.fori_loop(0, NSL // L, red_body, 0, unroll=4)
    pltpu.sync_copy(dsl_v, dinv_sh.at[pl.ds(nbase, NSL)])

    plsc.subcore_barrier()

    table_copies = (
        pltpu.make_async_copy(diag_sh, diag_t, sem_t),
        pltpu.make_async_copy(dinv_sh, dinv_t, sem_t),
    )
    for c in table_copies:
        c.start()
    for c in table_copies:
        c.wait()
    for c in edge_copies:
        c.wait()

    def edge_body(i, _):
        s = pl.ds(i * L, L)
        dv = plsc.load_gather(dinv_t, [row_v[s]])
        gv = plsc.load_gather(diag_t, [cole_v[s]])
        out_v[s] = ea_v[s] * dv * gv
        return 0

    with jax.named_scope("ph_edge"):
        lax.fori_loop(0, EC // L, edge_body, 0, unroll=8)
    pltpu.sync_copy(out_v, out_hbm.at[pl.ds(eoff, EC)])


def kernel(x, edge_index, edge_attr, W, b):
    w_flat = W.reshape(D)
    x_flat = x.reshape(N * D)
    ei_flat = edge_index.reshape(2 * E)
    b_vec = jnp.broadcast_to(b.reshape(1), (L,)).astype(jnp.float32)
    adj_val, _ = _node_attention_kernel(x_flat, ei_flat, edge_attr,
                                        w_flat, b_vec)
    return (edge_index, adj_val)


# speedup vs baseline: 1.1970x; 1.0026x over previous
import functools

import jax
import jax.numpy as jnp
from jax import lax
from jax.experimental import pallas as pl
from jax.experimental.pallas import tpu as pltpu
from jax.experimental.pallas import tpu_sc as plsc

N, E, D = 10000, 320000, 128
NC, NS = 2, 16
NW = NC * NS
L = 16
SLICE = 320
NSL = 640
EC = E // NW
ECA = E // NS
_MESH = plsc.VectorSubcoreMesh(core_axis_name="c", subcore_axis_name="s")
_PARAMS = pltpu.CompilerParams(needs_layout_passes=False)


@functools.partial(
    pl.kernel,
    out_type=(
        jax.ShapeDtypeStruct((N,), jnp.float32),
        jax.ShapeDtypeStruct((N,), jnp.float32),
    ),
    mesh=_MESH,
    compiler_params=_PARAMS,
    scratch_types=(
        pltpu.VMEM((SLICE * D,), jnp.float32),
        pltpu.VMEM((D,), jnp.float32),
        pltpu.VMEM((L,), jnp.float32),
        pltpu.VMEM((SLICE,), jnp.float32),
        pltpu.VMEM((ECA,), jnp.int32),
        pltpu.VMEM((N,), jnp.float32),
        pltpu.VMEM((NS * NSL,), jnp.float32),
        pltpu.VMEM((NSL,), jnp.float32),
        pltpu.VMEM_SHARED((NS * N,), jnp.float32),
        pltpu.SemaphoreType.DMA,
        pltpu.SemaphoreType.DMA,
    ),
)
def _diag_deg_kernel(x_hbm, ei_hbm, w_hbm, b_hbm, diag_hbm, dinv_hbm,
                     x_v, w_v, b_v, z_v, col_v, hist_v, hsl_v, dsl_v,
                     hist_sh, sem_x, sem_h):
    cid = lax.axis_index("c")
    sid = lax.axis_index("s")
    wid = cid * NS + sid
    base = pl.multiple_of(jnp.minimum(wid * SLICE, N - SLICE), 8)
    nbase = pl.multiple_of(jnp.minimum(sid * NSL, N - NSL), 8)

    hx = pltpu.async_copy(x_hbm.at[pl.ds(base * D, SLICE * D)], x_v, sem_x)
    pltpu.sync_copy(ei_hbm.at[pl.ds(E + sid * ECA, ECA)], col_v)
    pltpu.sync_copy(w_hbm, w_v)
    pltpu.sync_copy(b_hbm, b_v)

    def fill_zero(k, _):
        hist_v[pl.ds(k * L, L)] = jnp.zeros((L,), jnp.float32)
        return 0

    lax.fori_loop(0, N // L, fill_zero, 0, unroll=8)

    one16 = jnp.full((L,), 1.0, jnp.float32)

    def hist_body(k, _):
        plsc.addupdate_scatter(hist_v, [col_v[pl.ds(k * L, L)]], one16)
        return 0

    with jax.named_scope("ph_hist"):
        lax.fori_loop(0, ECA // L, hist_body, 0, unroll=8)

    with jax.named_scope("ph_stage"):
        pltpu.sync_copy(hist_v, hist_sh.at[pl.ds(sid * N, N)])
        plsc.subcore_barrier()

    slice_copies = tuple(
        pltpu.make_async_copy(
            hist_sh.at[pl.ds(t * N + nbase, NSL)],
            hsl_v.at[pl.ds(t * NSL, NSL)],
            sem_h,
        )
        for t in range(NS)
    )
    with jax.named_scope("ph_pull"):
        for c in slice_copies:
            c.start()
        for c in slice_copies:
            c.wait()

    def red_body(k, _):
        acc = [
            hsl_v[pl.ds(t * NSL + k * L, L)]
            + hsl_v[pl.ds((t + 1) * NSL + k * L, L)]
            for t in range(0, NS, 2)
        ]
        acc = [acc[t] + acc[t + 1] for t in range(0, 8, 2)]
        acc = [acc[t] + acc[t + 1] for t in range(0, 4, 2)]
        dsl_v[pl.ds(k * L, L)] = 1.0 / (acc[0] + acc[1])
        return 0

    with jax.named_scope("ph_reduce"):
        lax.fori_loop(0, NSL // L, red_body, 0, unroll=4)

    @pl.when(cid == 0)
    def _():
        pltpu.sync_copy(dsl_v, dinv_hbm.at[pl.ds(nbase, NSL)])

    hx.wait()
    wregs = [w_v[pl.ds(d8 * L, L)] for d8 in range(D // L)]

    lane = jnp.arange(L, dtype=jnp.int32)

    def group_body(g, _):
        zvec = jnp.zeros((L,), jnp.float32)
        for j in range(L):
            node = g * L + j
            acc0 = x_v[pl.ds(node * D, L)] * wregs[0]
            acc1 = x_v[pl.ds(node * D + L, L)] * wregs[1]
            for d8 in range(2, D // L, 2):
                acc0 = acc0 + x_v[pl.ds(node * D + d8 * L, L)] * wregs[d8]
                acc1 = acc1 + x_v[pl.ds(node * D + (d8 + 1) * L, L)] * wregs[d8 + 1]
            zvec = jnp.where(lane == j, jnp.sum(acc0 + acc1), zvec)
        z_v[pl.ds(g * L, L)] = zvec
        return 0

    with jax.named_scope("ph_dot"):
        lax.fori_loop(0, SLICE // L, group_body, 0)

    def sig_body(j, _):
        zv = z_v[pl.ds(j * L, L)] + b_v[...]
        z_v[pl.ds(j * L, L)] = 1.0 / (1.0 + jnp.exp(-zv))
        return 0

    lax.fori_loop(0, SLICE // L, sig_body, 0, unroll=4)
    pltpu.sync_copy(z_v, diag_hbm.at[pl.ds(base, SLICE)])


@functools.partial(
    pl.kernel,
    out_type=jax.ShapeDtypeStruct((E,), jnp.float32),
    mesh=_MESH,
    compiler_params=_PARAMS,
    scratch_types=(
        pltpu.VMEM((N,), jnp.float32),
        pltpu.VMEM((N,), jnp.float32),
        pltpu.VMEM((EC,), jnp.int32),
        pltpu.VMEM((EC,), jnp.int32),
        pltpu.VMEM((EC,), jnp.float32),
        pltpu.VMEM((EC,), jnp.float32),
        pltpu.SemaphoreType.DMA,
    ),
)
def _edge_kernel(ei_hbm, ea_hbm, diag_hbm, dinv_hbm, out_hbm,
                 diag_v, dinv_v, row_v, col_v, ea_v, out_v, sem):
    cid = lax.axis_index("c")
    sid = lax.axis_index("s")
    wid = cid * NS + sid
    off = wid * EC

    copies = (
        pltpu.make_async_copy(diag_hbm, diag_v, sem),
        pltpu.make_async_copy(dinv_hbm, dinv_v, sem),
        pltpu.make_async_copy(ei_hbm.at[pl.ds(off, EC)], row_v, sem),
        pltpu.make_async_copy(ei_hbm.at[pl.ds(E + off, EC)], col_v, sem),
        pltpu.make_async_copy(ea_hbm.at[pl.ds(off, EC)], ea_v, sem),
    )
    for c in copies:
        c.start()
    for c in copies:
        c.wait()

    def edge_body(i, _):
        s = pl.ds(i * L, L)
        dv = plsc.load_gather(dinv_v, [row_v[s]])
        gv = plsc.load_gather(diag_v, [col_v[s]])
        out_v[s] = ea_v[s] * dv * gv
        return 0

    lax.fori_loop(0, EC // L, edge_body, 0, unroll=8)
    pltpu.sync_copy(out_v, out_hbm.at[pl.ds(off, EC)])


def kernel(x, edge_index, edge_attr, W, b):
    w_flat = W.reshape(D)
    x_flat = x.reshape(N * D)
    ei_flat = edge_index.reshape(2 * E)
    b_vec = jnp.broadcast_to(b.reshape(1), (L,)).astype(jnp.float32)
    diag, dinv = _diag_deg_kernel(x_flat, ei_flat, w_flat, b_vec)
    adj_val = _edge_kernel(ei_flat, edge_attr, diag, dinv)
    return (edge_index, adj_val)
